# baseline (speedup 1.0000x reference)
import functools
import jax
import jax.numpy as jnp
from jax import lax
from jax.experimental import pallas as pl
from jax.experimental.pallas import tpu as pltpu
from jax.experimental.pallas import tpu_sc as plsc

B = 16
N = 16384
NP1, NS1 = 128, 32
NP2, NS2 = 32, 64
BIGI = 1 << 30
INF = 1e30
SLANES = 256


def _fps1_body(xp_ref, yp_ref, zp_ref, o_ref):
    xp, yp, zp = xp_ref[...], yp_ref[...], zp_ref[...]
    col = lax.broadcasted_iota(jnp.int32, (B, N), 1)
    ksel_iota = lax.broadcasted_iota(jnp.int32, (B, NP1), 1)

    def step(k, state):
        dist, far, cx, cy, cz = state
        m = col == far
        cxs = jnp.sum(jnp.where(m, xp, 0.0), axis=1, keepdims=True)
        cys = jnp.sum(jnp.where(m, yp, 0.0), axis=1, keepdims=True)
        czs = jnp.sum(jnp.where(m, zp, 0.0), axis=1, keepdims=True)
        d = (xp - cxs) ** 2 + (yp - cys) ** 2 + (zp - czs) ** 2
        dist = jnp.minimum(dist, d)
        mx = jnp.max(dist, axis=1, keepdims=True)
        nf = jnp.min(jnp.where(dist == mx, col, BIGI), axis=1, keepdims=True)
        ksel = ksel_iota == k
        return (dist, nf, jnp.where(ksel, cxs, cx), jnp.where(ksel, cys, cy),
                jnp.where(ksel, czs, cz))

    init = (jnp.full((B, N), 1e10, jnp.float32), jnp.zeros((B, 1), jnp.int32),
            jnp.zeros((B, NP1), jnp.float32), jnp.zeros((B, NP1), jnp.float32),
            jnp.zeros((B, NP1), jnp.float32))
    _, _, cx, cy, cz = lax.fori_loop(0, NP1, step, init)
    o_ref[...] = jnp.concatenate(
        [cx[:, None, :], cy[:, None, :], cz[:, None, :],
         jnp.zeros((B, 5, NP1), jnp.float32)], axis=1)


def _fps1(xp, yp, zp):
    return pl.pallas_call(
        _fps1_body,
        out_shape=jax.ShapeDtypeStruct((B, 8, NP1), jnp.float32),
    )(xp, yp, zp)


def _knn1_body(nx_ref, pts_ref, o_ref):
    b = pl.program_id(0)
    nx = nx_ref[0][0:3, :]
    pts = pts_ref[0]
    a2 = jnp.sum(nx * nx, axis=0, keepdims=True)
    b2 = jnp.sum(pts * pts, axis=0, keepdims=True)
    nx4 = jnp.concatenate([-2.0 * nx, a2], axis=0)
    p4 = jnp.concatenate([pts, jnp.ones((1, N), jnp.float32)], axis=0)
    d = lax.dot_general(nx4, p4, (((0,), (0,)), ((), ())),
                        preferred_element_type=jnp.float32) + b2
    col = lax.broadcasted_iota(jnp.int32, (NP1, N), 1)
    ksel_iota = lax.broadcasted_iota(jnp.int32, (NP1, NS1), 1)

    def step(k, state):
        dcur, idxout = state
        m = jnp.min(dcur, axis=1, keepdims=True)
        idx = jnp.min(jnp.where(dcur == m, col, BIGI), axis=1, keepdims=True)
        knock = col == idx
        dcur = jnp.where(knock, INF, dcur)
        idxout = jnp.where(ksel_iota == k, idx, idxout)
        return dcur, idxout

    _, idxout = lax.fori_loop(0, NS1, step, (d, jnp.zeros((NP1, NS1), jnp.int32)))
    o_ref[...] = (idxout + b * N)[None]


def _knn1(nxyz1, pts_t):
    return pl.pallas_call(
        _knn1_body,
        grid=(B,),
        compiler_params=pltpu.CompilerParams(dimension_semantics=("parallel",)),
        in_specs=[
            pl.BlockSpec((1, 8, NP1), lambda b: (b, 0, 0)),
            pl.BlockSpec((1, 3, N), lambda b: (b, 0, 0)),
        ],
        out_specs=pl.BlockSpec((1, NP1, NS1), lambda b: (b, 0, 0)),
        out_shape=jax.ShapeDtypeStruct((B, NP1, NS1), jnp.int32),
    )(nxyz1, pts_t)


_SC_NC, _SC_NS = 2, 16
_NW = _SC_NC * _SC_NS
_NG = B * NP1 * NS1
_GPW = _NG // _NW
_CH = 512
_TD = 128


def _sc_gather(table, idx_flat):
    mesh = plsc.VectorSubcoreMesh(core_axis_name="c", subcore_axis_name="s")

    @functools.partial(
        pl.kernel,
        out_type=jax.ShapeDtypeStruct((_NG, _TD), jnp.float32),
        mesh=mesh,
        scratch_types=[
            pltpu.VMEM((_CH,), jnp.int32),
            pltpu.VMEM((_CH, _TD), jnp.float32),
            pltpu.SemaphoreType.DMA,
        ],
    )
    def k(table_hbm, idx_hbm, out_hbm, idx_v, rows_v, sem):
        wid = lax.axis_index("s") * _SC_NC + lax.axis_index("c")
        base = wid * _GPW
        for c in range(_GPW // _CH):
            off = base + c * _CH
            pltpu.sync_copy(idx_hbm.at[pl.ds(off, _CH)], idx_v)
            pltpu.async_copy(table_hbm.at[idx_v], rows_v, sem).wait()
            pltpu.sync_copy(rows_v, out_hbm.at[pl.ds(off, _CH)])

    return k(table, idx_flat)


def _stats_mat(y, cout):
    s = jnp.sum(y, axis=0)
    ss = jnp.sum(y * y, axis=0)
    if cout < SLANES:
        pad = jnp.zeros((SLANES - cout,), jnp.float32)
        s = jnp.concatenate([s, pad])
        ss = jnp.concatenate([ss, pad])
    sp = s[None]
    ssp = ss[None]
    return jnp.concatenate([sp, ssp, jnp.zeros((6, SLANES), jnp.float32)], axis=0)


def _bn(y, st, cout, cnt):
    ssum = jnp.sum(st[:, 0, :], axis=0)[:cout]
    sss = jnp.sum(st[:, 1, :], axis=0)[:cout]
    mean = ssum / cnt
    var = sss / cnt - mean * mean
    inv = lax.rsqrt(var + 1e-5)
    return jnp.maximum((y - mean[None, :]) * inv[None, :], 0.0)


def _c1_body(g_ref, nx_ref, w_ref, b_ref, oy_ref, os_ref):
    g = g_ref[0]
    nx = nx_ref[0][0:3, :]
    w = w_ref[...]
    y = lax.dot_general(g, w, (((1,), (1,)), ((), ())),
                        preferred_element_type=jnp.float32)
    corr = lax.dot_general(nx, w[:, 0:3], (((0,), (1,)), ((), ())),
                           preferred_element_type=jnp.float32)
    y = (y.reshape(NP1, NS1, 64) - corr[:, None, :] + b_ref[0][None, None, :])
    y = y.reshape(NP1 * NS1, 64)
    oy_ref[0] = y
    os_ref[0] = _stats_mat(y, 64)


def _c1(g, nxyz1, w1p, b1r):
    return pl.pallas_call(
        _c1_body,
        grid=(B,),
        compiler_params=pltpu.CompilerParams(dimension_semantics=("parallel",)),
        in_specs=[
            pl.BlockSpec((1, NP1 * NS1, _TD), lambda b: (b, 0, 0)),
            pl.BlockSpec((1, 8, NP1), lambda b: (b, 0, 0)),
            pl.BlockSpec((64, _TD), lambda b: (0, 0)),
            pl.BlockSpec((1, 64), lambda b: (0, 0)),
        ],
        out_specs=[
            pl.BlockSpec((1, NP1 * NS1, 64), lambda b: (b, 0, 0)),
            pl.BlockSpec((1, 8, SLANES), lambda b: (b, 0, 0)),
        ],
        out_shape=[
            jax.ShapeDtypeStruct((B, NP1 * NS1, 64), jnp.float32),
            jax.ShapeDtypeStruct((B, 8, SLANES), jnp.float32),
        ],
    )(g, nxyz1, w1p, b1r)


def _conv_bn(y_prev, st_prev, w, br, cin, cout, p):
    cnt = float(B * p)

    def body(y_ref, st_ref, w_ref, b_ref, oy_ref, os_ref):
        x = _bn(y_ref[0], st_ref[...], cin, cnt)
        y = lax.dot_general(x, w_ref[...], (((1,), (1,)), ((), ())),
                            preferred_element_type=jnp.float32) + b_ref[0][None, :]
        oy_ref[0] = y
        os_ref[0] = _stats_mat(y, cout)

    return pl.pallas_call(
        body,
        grid=(B,),
        compiler_params=pltpu.CompilerParams(dimension_semantics=("parallel",)),
        in_specs=[
            pl.BlockSpec((1, p, cin), lambda b: (b, 0, 0)),
            pl.BlockSpec((B, 8, SLANES), lambda b: (0, 0, 0)),
            pl.BlockSpec((cout, cin), lambda b: (0, 0)),
            pl.BlockSpec((1, cout), lambda b: (0, 0)),
        ],
        out_specs=[
            pl.BlockSpec((1, p, cout), lambda b: (b, 0, 0)),
            pl.BlockSpec((1, 8, SLANES), lambda b: (b, 0, 0)),
        ],
        out_shape=[
            jax.ShapeDtypeStruct((B, p, cout), jnp.float32),
            jax.ShapeDtypeStruct((B, 8, SLANES), jnp.float32),
        ],
    )(y_prev, st_prev, w, br)


def _pool1_body(y_ref, st_ref, o_ref):
    x = _bn(y_ref[0], st_ref[...], 128, float(B * NP1 * NS1))
    o_ref[0] = jnp.max(x.reshape(NP1, NS1, 128), axis=1)


def _pool1(y3, st3):
    return pl.pallas_call(
        _pool1_body,
        grid=(B,),
        compiler_params=pltpu.CompilerParams(dimension_semantics=("parallel",)),
        in_specs=[
            pl.BlockSpec((1, NP1 * NS1, 128), lambda b: (b, 0, 0)),
            pl.BlockSpec((B, 8, SLANES), lambda b: (0, 0, 0)),
        ],
        out_specs=pl.BlockSpec((1, NP1, 128), lambda b: (b, 0, 0)),
        out_shape=jax.ShapeDtypeStruct((B, NP1, 128), jnp.float32),
    )(y3, st3)


def _sa2a_body(f_ref, nx_ref, w_ref, b_ref, oy_ref, os_ref):
    f1 = f_ref[0]
    nb = nx_ref[0]
    cxr, cyr, czr = nb[0:1, :], nb[1:2, :], nb[2:3, :]
    col1 = lax.broadcasted_iota(jnp.int32, (1, NP1), 1)
    kiota = lax.broadcasted_iota(jnp.int32, (NP2, 1), 0)

    def fstep(k, state):
        dist, far, sx, sy, sz = state
        m = col1 == far
        cxs = jnp.sum(jnp.where(m, cxr, 0.0), axis=1, keepdims=True)
        cys = jnp.sum(jnp.where(m, cyr, 0.0), axis=1, keepdims=True)
        czs = jnp.sum(jnp.where(m, czr, 0.0), axis=1, keepdims=True)
        d = (cxr - cxs) ** 2 + (cyr - cys) ** 2 + (czr - czs) ** 2
        dist = jnp.minimum(dist, d)
        mx = jnp.max(dist, axis=1, keepdims=True)
        nf = jnp.min(jnp.where(dist == mx, col1, BIGI), axis=1, keepdims=True)
        ks = kiota == k
        return (dist, nf, jnp.where(ks, cxs, sx), jnp.where(ks, cys, sy),
                jnp.where(ks, czs, sz))

    finit = (jnp.full((1, NP1), 1e10, jnp.float32), jnp.zeros((1, 1), jnp.int32),
             jnp.zeros((NP2, 1), jnp.float32), jnp.zeros((NP2, 1), jnp.float32),
             jnp.zeros((NP2, 1), jnp.float32))
    _, _, sx, sy, sz = lax.fori_loop(0, NP2, fstep, finit)

    d2 = (sx - cxr) ** 2 + (sy - cyr) ** 2 + (sz - czr) ** 2
    col2 = lax.broadcasted_iota(jnp.int32, (NP2, NP1), 1)
    siota = lax.broadcasted_iota(jnp.int32, (1, NS2, 1), 1)

    def kstep(k, state):
        dcur, acc = state
        m = jnp.min(dcur, axis=1, keepdims=True)
        idx = jnp.min(jnp.where(dcur == m, col2, BIGI), axis=1, keepdims=True)
        knock = col2 == idx
        acc = jnp.where(siota == k, knock[:, None, :].astype(jnp.float32), acc)
        return jnp.where(knock, INF, dcur), acc

    _, acc = lax.fori_loop(
        0, NS2, kstep, (d2, jnp.zeros((NP2, NS2, NP1), jnp.float32)))
    oh = acc.reshape(NP2 * NS2, NP1)

    gx = lax.dot_general(oh, cxr, (((1,), (1,)), ((), ())),
                         preferred_element_type=jnp.float32)
    gy = lax.dot_general(oh, cyr, (((1,), (1,)), ((), ())),
                         preferred_element_type=jnp.float32)
    gz = lax.dot_general(oh, czr, (((1,), (1,)), ((), ())),
                         preferred_element_type=jnp.float32)
    gf = lax.dot_general(oh, f1, (((1,), (1,)), ((), ())),
                         preferred_element_type=jnp.float32)
    g = jnp.concatenate([gx, gy, gz, gf], axis=1)
    s = jnp.concatenate([sx, sy, sz, jnp.zeros((NP2, 128), jnp.float32)],
                        axis=1)
    g = (g.reshape(NP2, NS2, 131) - s[:, None, :]).reshape(NP2 * NS2, 131)
    y = lax.dot_general(g, w_ref[...], (((1,), (1,)), ((), ())),
                        preferred_element_type=jnp.float32) + b_ref[0][None, :]
    oy_ref[0] = y
    os_ref[0] = _stats_mat(y, 128)


def _sa2a(f1, nxyz1, w4, b4r):
    return pl.pallas_call(
        _sa2a_body,
        grid=(B,),
        compiler_params=pltpu.CompilerParams(dimension_semantics=("parallel",)),
        in_specs=[
            pl.BlockSpec((1, NP1, 128), lambda b: (b, 0, 0)),
            pl.BlockSpec((1, 8, NP1), lambda b: (b, 0, 0)),
            pl.BlockSpec((128, 131), lambda b: (0, 0)),
            pl.BlockSpec((1, 128), lambda b: (0, 0)),
        ],
        out_specs=[
            pl.BlockSpec((1, NP2 * NS2, 128), lambda b: (b, 0, 0)),
            pl.BlockSpec((1, 8, SLANES), lambda b: (b, 0, 0)),
        ],
        out_shape=[
            jax.ShapeDtypeStruct((B, NP2 * NS2, 128), jnp.float32),
            jax.ShapeDtypeStruct((B, 8, SLANES), jnp.float32),
        ],
    )(f1, nxyz1, w4, b4r)


def _fin_body(y_ref, st_ref, o_ref):
    x = _bn(y_ref[0], st_ref[...], 256, float(B * NP2 * NS2))
    o_ref[0] = jnp.max(x.reshape(NP2, NS2, 256), axis=(1, 2)).reshape(1, NP2)


def _fin(y6, st6):
    return pl.pallas_call(
        _fin_body,
        grid=(B,),
        compiler_params=pltpu.CompilerParams(dimension_semantics=("parallel",)),
        in_specs=[
            pl.BlockSpec((1, NP2 * NS2, 256), lambda b: (b, 0, 0)),
            pl.BlockSpec((B, 8, SLANES), lambda b: (0, 0, 0)),
        ],
        out_specs=pl.BlockSpec((1, 1, NP2), lambda b: (b, 0, 0)),
        out_shape=jax.ShapeDtypeStruct((B, 1, NP2), jnp.float32),
    )(y6, st6)


@jax.jit
def kernel(points, W1, b1, W2, b2, W3, b3, W4, b4, W5, b5, W6, b6):
    xp, yp, zp = points[..., 0], points[..., 1], points[..., 2]
    nxyz1 = _fps1(xp, yp, zp)
    pts_t = jnp.transpose(points[..., :3], (0, 2, 1))
    idx1 = _knn1(nxyz1, pts_t)
    table = jnp.pad(points.reshape(B * N, 6), ((0, 0), (0, _TD - 6)))
    g = _sc_gather(table, idx1.reshape(_NG))
    g = g.reshape(B, NP1 * NS1, _TD)

    w1p = jnp.pad(W1, ((0, 0), (0, _TD - 6)))
    y1, s1 = _c1(g, nxyz1, w1p, b1[None])
    y2, s2 = _conv_bn(y1, s1, W2, b2[None], 64, 64, NP1 * NS1)
    y3, s3 = _conv_bn(y2, s2, W3, b3[None], 64, 128, NP1 * NS1)
    f1 = _pool1(y3, s3)

    y4, s4 = _sa2a(f1, nxyz1, W4, b4[None])
    y5, s5 = _conv_bn(y4, s4, W5, b5[None], 128, 128, NP2 * NS2)
    y6, s6 = _conv_bn(y5, s5, W6, b6[None], 128, 256, NP2 * NS2)
    return _fin(y6, s6).reshape(B, NP2)


# speedup vs baseline: 1.0379x; 1.0379x over previous
import functools
import jax
import jax.numpy as jnp
from jax import lax
from jax.experimental import pallas as pl
from jax.experimental.pallas import tpu as pltpu
from jax.experimental.pallas import tpu_sc as plsc

B = 16
N = 16384
NP1, NS1 = 128, 32
NP2, NS2 = 32, 64
BIGI = 1 << 30
INF = 1e30
SLANES = 256


_FH = 8


def _fps1_body(xp_ref, yp_ref, zp_ref, o_ref):
    xp, yp, zp = xp_ref[...], yp_ref[...], zp_ref[...]
    col = lax.broadcasted_iota(jnp.int32, (_FH, N), 1)
    ksel_iota = lax.broadcasted_iota(jnp.int32, (_FH, NP1), 1)

    def step(k, state):
        dist, far, cx, cy, cz = state
        m = col == far
        cxs = jnp.sum(jnp.where(m, xp, 0.0), axis=1, keepdims=True)
        cys = jnp.sum(jnp.where(m, yp, 0.0), axis=1, keepdims=True)
        czs = jnp.sum(jnp.where(m, zp, 0.0), axis=1, keepdims=True)
        d = (xp - cxs) ** 2 + (yp - cys) ** 2 + (zp - czs) ** 2
        dist = jnp.minimum(dist, d)
        mx = jnp.max(dist, axis=1, keepdims=True)
        nf = jnp.min(jnp.where(dist == mx, col, BIGI), axis=1, keepdims=True)
        ksel = ksel_iota == k
        return (dist, nf, jnp.where(ksel, cxs, cx), jnp.where(ksel, cys, cy),
                jnp.where(ksel, czs, cz))

    init = (jnp.full((_FH, N), 1e10, jnp.float32), jnp.zeros((_FH, 1), jnp.int32),
            jnp.zeros((_FH, NP1), jnp.float32), jnp.zeros((_FH, NP1), jnp.float32),
            jnp.zeros((_FH, NP1), jnp.float32))
    _, _, cx, cy, cz = lax.fori_loop(0, NP1, step, init)
    o_ref[...] = jnp.concatenate(
        [cx[:, None, :], cy[:, None, :], cz[:, None, :],
         jnp.zeros((_FH, 5, NP1), jnp.float32)], axis=1)


def _fps1(xp, yp, zp):
    return pl.pallas_call(
        _fps1_body,
        grid=(B // _FH,),
        compiler_params=pltpu.CompilerParams(dimension_semantics=("parallel",)),
        in_specs=[pl.BlockSpec((_FH, N), lambda i: (i, 0))] * 3,
        out_specs=pl.BlockSpec((_FH, 8, NP1), lambda i: (i, 0, 0)),
        out_shape=jax.ShapeDtypeStruct((B, 8, NP1), jnp.float32),
    )(xp, yp, zp)


def _knn1_body(nx_ref, pts_ref, o_ref):
    b = pl.program_id(0)
    nx = nx_ref[0][0:3, :]
    pts = pts_ref[0]
    a2 = jnp.sum(nx * nx, axis=0, keepdims=True)
    b2 = jnp.sum(pts * pts, axis=0, keepdims=True)
    nx4 = jnp.concatenate([-2.0 * nx, a2], axis=0)
    p4 = jnp.concatenate([pts, jnp.ones((1, N), jnp.float32)], axis=0)
    d = lax.dot_general(nx4, p4, (((0,), (0,)), ((), ())),
                        preferred_element_type=jnp.float32) + b2
    col = lax.broadcasted_iota(jnp.int32, (NP1, N), 1)
    ksel_iota = lax.broadcasted_iota(jnp.int32, (NP1, NS1), 1)

    def step(k, state):
        mprev, iprev, idxout = state
        pred = (d > mprev) | ((d == mprev) & (col > iprev))
        m = jnp.min(jnp.where(pred, d, INF), axis=1, keepdims=True)
        idx = jnp.min(jnp.where(pred & (d == m), col, BIGI), axis=1,
                      keepdims=True)
        idxout = jnp.where(ksel_iota == k, idx, idxout)
        return m, idx, idxout

    init = (jnp.full((NP1, 1), -INF, jnp.float32),
            jnp.full((NP1, 1), -1, jnp.int32),
            jnp.zeros((NP1, NS1), jnp.int32))
    _, _, idxout = lax.fori_loop(0, NS1, step, init)
    o_ref[...] = (idxout + b * N)[None]


def _knn1(nxyz1, pts_t):
    return pl.pallas_call(
        _knn1_body,
        grid=(B,),
        compiler_params=pltpu.CompilerParams(dimension_semantics=("parallel",)),
        in_specs=[
            pl.BlockSpec((1, 8, NP1), lambda b: (b, 0, 0)),
            pl.BlockSpec((1, 3, N), lambda b: (b, 0, 0)),
        ],
        out_specs=pl.BlockSpec((1, NP1, NS1), lambda b: (b, 0, 0)),
        out_shape=jax.ShapeDtypeStruct((B, NP1, NS1), jnp.int32),
    )(nxyz1, pts_t)


_SC_NC, _SC_NS = 2, 16
_NW = _SC_NC * _SC_NS
_NG = B * NP1 * NS1
_GPW = _NG // _NW
_CH = 512
_TD = 128


def _sc_gather(table, idx_flat):
    mesh = plsc.VectorSubcoreMesh(core_axis_name="c", subcore_axis_name="s")

    @functools.partial(
        pl.kernel,
        out_type=jax.ShapeDtypeStruct((_NG, _TD), jnp.float32),
        mesh=mesh,
        scratch_types=[
            pltpu.VMEM((_CH,), jnp.int32),
            pltpu.VMEM((_CH, _TD), jnp.float32),
            pltpu.SemaphoreType.DMA,
        ],
    )
    def k(table_hbm, idx_hbm, out_hbm, idx_v, rows_v, sem):
        wid = lax.axis_index("s") * _SC_NC + lax.axis_index("c")
        base = wid * _GPW
        for c in range(_GPW // _CH):
            off = base + c * _CH
            pltpu.sync_copy(idx_hbm.at[pl.ds(off, _CH)], idx_v)
            pltpu.async_copy(table_hbm.at[idx_v], rows_v, sem).wait()
            pltpu.sync_copy(rows_v, out_hbm.at[pl.ds(off, _CH)])

    return k(table, idx_flat)


def _stats_mat(y, cout):
    s = jnp.sum(y, axis=0)
    ss = jnp.sum(y * y, axis=0)
    if cout < SLANES:
        pad = jnp.zeros((SLANES - cout,), jnp.float32)
        s = jnp.concatenate([s, pad])
        ss = jnp.concatenate([ss, pad])
    sp = s[None]
    ssp = ss[None]
    return jnp.concatenate([sp, ssp, jnp.zeros((6, SLANES), jnp.float32)], axis=0)


def _bn(y, st, cout, cnt):
    ssum = jnp.sum(st[:, 0, :], axis=0)[:cout]
    sss = jnp.sum(st[:, 1, :], axis=0)[:cout]
    mean = ssum / cnt
    var = sss / cnt - mean * mean
    inv = lax.rsqrt(var + 1e-5)
    return jnp.maximum((y - mean[None, :]) * inv[None, :], 0.0)


def _c1_body(g_ref, nx_ref, w_ref, b_ref, oy_ref, os_ref):
    g = g_ref[0]
    nx = nx_ref[0][0:3, :]
    w = w_ref[...]
    y = lax.dot_general(g, w, (((1,), (1,)), ((), ())),
                        preferred_element_type=jnp.float32)
    corr = lax.dot_general(nx, w[:, 0:3], (((0,), (1,)), ((), ())),
                           preferred_element_type=jnp.float32)
    y = (y.reshape(NP1, NS1, 64) - corr[:, None, :] + b_ref[0][None, None, :])
    y = y.reshape(NP1 * NS1, 64)
    oy_ref[0] = y
    os_ref[0] = _stats_mat(y, 64)


def _c1(g, nxyz1, w1p, b1r):
    return pl.pallas_call(
        _c1_body,
        grid=(B,),
        compiler_params=pltpu.CompilerParams(dimension_semantics=("parallel",)),
        in_specs=[
            pl.BlockSpec((1, NP1 * NS1, _TD), lambda b: (b, 0, 0)),
            pl.BlockSpec((1, 8, NP1), lambda b: (b, 0, 0)),
            pl.BlockSpec((64, _TD), lambda b: (0, 0)),
            pl.BlockSpec((1, 64), lambda b: (0, 0)),
        ],
        out_specs=[
            pl.BlockSpec((1, NP1 * NS1, 64), lambda b: (b, 0, 0)),
            pl.BlockSpec((1, 8, SLANES), lambda b: (b, 0, 0)),
        ],
        out_shape=[
            jax.ShapeDtypeStruct((B, NP1 * NS1, 64), jnp.float32),
            jax.ShapeDtypeStruct((B, 8, SLANES), jnp.float32),
        ],
    )(g, nxyz1, w1p, b1r)


def _conv_bn(y_prev, st_prev, w, br, cin, cout, p):
    cnt = float(B * p)

    def body(y_ref, st_ref, w_ref, b_ref, oy_ref, os_ref):
        x = _bn(y_ref[0], st_ref[...], cin, cnt)
        y = lax.dot_general(x, w_ref[...], (((1,), (1,)), ((), ())),
                            preferred_element_type=jnp.float32) + b_ref[0][None, :]
        oy_ref[0] = y
        os_ref[0] = _stats_mat(y, cout)

    return pl.pallas_call(
        body,
        grid=(B,),
        compiler_params=pltpu.CompilerParams(dimension_semantics=("parallel",)),
        in_specs=[
            pl.BlockSpec((1, p, cin), lambda b: (b, 0, 0)),
            pl.BlockSpec((B, 8, SLANES), lambda b: (0, 0, 0)),
            pl.BlockSpec((cout, cin), lambda b: (0, 0)),
            pl.BlockSpec((1, cout), lambda b: (0, 0)),
        ],
        out_specs=[
            pl.BlockSpec((1, p, cout), lambda b: (b, 0, 0)),
            pl.BlockSpec((1, 8, SLANES), lambda b: (b, 0, 0)),
        ],
        out_shape=[
            jax.ShapeDtypeStruct((B, p, cout), jnp.float32),
            jax.ShapeDtypeStruct((B, 8, SLANES), jnp.float32),
        ],
    )(y_prev, st_prev, w, br)


def _pool1_body(y_ref, st_ref, o_ref):
    x = _bn(y_ref[0], st_ref[...], 128, float(B * NP1 * NS1))
    o_ref[0] = jnp.max(x.reshape(NP1, NS1, 128), axis=1)


def _pool1(y3, st3):
    return pl.pallas_call(
        _pool1_body,
        grid=(B,),
        compiler_params=pltpu.CompilerParams(dimension_semantics=("parallel",)),
        in_specs=[
            pl.BlockSpec((1, NP1 * NS1, 128), lambda b: (b, 0, 0)),
            pl.BlockSpec((B, 8, SLANES), lambda b: (0, 0, 0)),
        ],
        out_specs=pl.BlockSpec((1, NP1, 128), lambda b: (b, 0, 0)),
        out_shape=jax.ShapeDtypeStruct((B, NP1, 128), jnp.float32),
    )(y3, st3)


def _sa2a_body(f_ref, nx_ref, w_ref, b_ref, oy_ref, os_ref):
    f1 = f_ref[0]
    nb = nx_ref[0]
    cxr, cyr, czr = nb[0:1, :], nb[1:2, :], nb[2:3, :]
    col1 = lax.broadcasted_iota(jnp.int32, (1, NP1), 1)
    kiota = lax.broadcasted_iota(jnp.int32, (NP2, 1), 0)

    def fstep(k, state):
        dist, far, sx, sy, sz = state
        m = col1 == far
        cxs = jnp.sum(jnp.where(m, cxr, 0.0), axis=1, keepdims=True)
        cys = jnp.sum(jnp.where(m, cyr, 0.0), axis=1, keepdims=True)
        czs = jnp.sum(jnp.where(m, czr, 0.0), axis=1, keepdims=True)
        d = (cxr - cxs) ** 2 + (cyr - cys) ** 2 + (czr - czs) ** 2
        dist = jnp.minimum(dist, d)
        mx = jnp.max(dist, axis=1, keepdims=True)
        nf = jnp.min(jnp.where(dist == mx, col1, BIGI), axis=1, keepdims=True)
        ks = kiota == k
        return (dist, nf, jnp.where(ks, cxs, sx), jnp.where(ks, cys, sy),
                jnp.where(ks, czs, sz))

    finit = (jnp.full((1, NP1), 1e10, jnp.float32), jnp.zeros((1, 1), jnp.int32),
             jnp.zeros((NP2, 1), jnp.float32), jnp.zeros((NP2, 1), jnp.float32),
             jnp.zeros((NP2, 1), jnp.float32))
    _, _, sx, sy, sz = lax.fori_loop(0, NP2, fstep, finit)

    d2 = (sx - cxr) ** 2 + (sy - cyr) ** 2 + (sz - czr) ** 2
    col2 = lax.broadcasted_iota(jnp.int32, (NP2, NP1), 1)
    siota = lax.broadcasted_iota(jnp.int32, (1, NS2, 1), 1)

    def kstep(k, state):
        dcur, acc = state
        m = jnp.min(dcur, axis=1, keepdims=True)
        idx = jnp.min(jnp.where(dcur == m, col2, BIGI), axis=1, keepdims=True)
        knock = col2 == idx
        acc = jnp.where(siota == k, knock[:, None, :].astype(jnp.float32), acc)
        return jnp.where(knock, INF, dcur), acc

    _, acc = lax.fori_loop(
        0, NS2, kstep, (d2, jnp.zeros((NP2, NS2, NP1), jnp.float32)))
    oh = acc.reshape(NP2 * NS2, NP1)

    gx = lax.dot_general(oh, cxr, (((1,), (1,)), ((), ())),
                         preferred_element_type=jnp.float32)
    gy = lax.dot_general(oh, cyr, (((1,), (1,)), ((), ())),
                         preferred_element_type=jnp.float32)
    gz = lax.dot_general(oh, czr, (((1,), (1,)), ((), ())),
                         preferred_element_type=jnp.float32)
    gf = lax.dot_general(oh, f1, (((1,), (1,)), ((), ())),
                         preferred_element_type=jnp.float32)
    g = jnp.concatenate([gx, gy, gz, gf], axis=1)
    s = jnp.concatenate([sx, sy, sz, jnp.zeros((NP2, 128), jnp.float32)],
                        axis=1)
    g = (g.reshape(NP2, NS2, 131) - s[:, None, :]).reshape(NP2 * NS2, 131)
    y = lax.dot_general(g, w_ref[...], (((1,), (1,)), ((), ())),
                        preferred_element_type=jnp.float32) + b_ref[0][None, :]
    oy_ref[0] = y
    os_ref[0] = _stats_mat(y, 128)


def _sa2a(f1, nxyz1, w4, b4r):
    return pl.pallas_call(
        _sa2a_body,
        grid=(B,),
        compiler_params=pltpu.CompilerParams(dimension_semantics=("parallel",)),
        in_specs=[
            pl.BlockSpec((1, NP1, 128), lambda b: (b, 0, 0)),
            pl.BlockSpec((1, 8, NP1), lambda b: (b, 0, 0)),
            pl.BlockSpec((128, 131), lambda b: (0, 0)),
            pl.BlockSpec((1, 128), lambda b: (0, 0)),
        ],
        out_specs=[
            pl.BlockSpec((1, NP2 * NS2, 128), lambda b: (b, 0, 0)),
            pl.BlockSpec((1, 8, SLANES), lambda b: (b, 0, 0)),
        ],
        out_shape=[
            jax.ShapeDtypeStruct((B, NP2 * NS2, 128), jnp.float32),
            jax.ShapeDtypeStruct((B, 8, SLANES), jnp.float32),
        ],
    )(f1, nxyz1, w4, b4r)


def _fin_body(y_ref, st_ref, o_ref):
    x = _bn(y_ref[0], st_ref[...], 256, float(B * NP2 * NS2))
    o_ref[0] = jnp.max(x.reshape(NP2, NS2, 256), axis=(1, 2)).reshape(1, NP2)


def _fin(y6, st6):
    return pl.pallas_call(
        _fin_body,
        grid=(B,),
        compiler_params=pltpu.CompilerParams(dimension_semantics=("parallel",)),
        in_specs=[
            pl.BlockSpec((1, NP2 * NS2, 256), lambda b: (b, 0, 0)),
            pl.BlockSpec((B, 8, SLANES), lambda b: (0, 0, 0)),
        ],
        out_specs=pl.BlockSpec((1, 1, NP2), lambda b: (b, 0, 0)),
        out_shape=jax.ShapeDtypeStruct((B, 1, NP2), jnp.float32),
    )(y6, st6)


@jax.jit
def kernel(points, W1, b1, W2, b2, W3, b3, W4, b4, W5, b5, W6, b6):
    xp, yp, zp = points[..., 0], points[..., 1], points[..., 2]
    nxyz1 = _fps1(xp, yp, zp)
    pts_t = jnp.transpose(points[..., :3], (0, 2, 1))
    idx1 = _knn1(nxyz1, pts_t)
    table = jnp.pad(points.reshape(B * N, 6), ((0, 0), (0, _TD - 6)))
    g = _sc_gather(table, idx1.reshape(_NG))
    g = g.reshape(B, NP1 * NS1, _TD)

    w1p = jnp.pad(W1, ((0, 0), (0, _TD - 6)))
    y1, s1 = _c1(g, nxyz1, w1p, b1[None])
    y2, s2 = _conv_bn(y1, s1, W2, b2[None], 64, 64, NP1 * NS1)
    y3, s3 = _conv_bn(y2, s2, W3, b3[None], 64, 128, NP1 * NS1)
    f1 = _pool1(y3, s3)

    y4, s4 = _sa2a(f1, nxyz1, W4, b4[None])
    y5, s5 = _conv_bn(y4, s4, W5, b5[None], 128, 128, NP2 * NS2)
    y6, s6 = _conv_bn(y5, s5, W6, b6[None], 128, 256, NP2 * NS2)
    return _fin(y6, s6).reshape(B, NP2)
